# baseline (speedup 1.0000x reference)
import functools

import jax
import jax.numpy as jnp
from jax import lax
from jax.experimental import pallas as pl
from jax.experimental.pallas import tpu as pltpu
from jax.experimental.pallas import tpu_sc as plsc

NUM_GRAPHS = 256
N_NODES = 100000
D_FEAT = 128

NC = 2
NS = 16
DC = D_FEAT // NC
B = 128
NBUF = 4

ROWS_PER_SUB = 6256
LAST_ROWS = N_NODES - (NS - 1) * ROWS_PER_SUB
FULL_BLOCKS = 48
TAIL_MAIN = ROWS_PER_SUB - FULL_BLOCKS * B
TAIL_LAST = LAST_ROWS - FULL_BLOCKS * B
GROWS = NUM_GRAPHS // NS


@functools.partial(
    pl.kernel,
    mesh=plsc.VectorSubcoreMesh(core_axis_name="c", subcore_axis_name="s"),
    out_type=jax.ShapeDtypeStruct((NUM_GRAPHS, D_FEAT), jnp.float32),
    compiler_params=pltpu.CompilerParams(use_tc_tiling_on_sc=False),
    scratch_types=[
        pltpu.VMEM((FULL_BLOCKS, B), jnp.int32),
        pltpu.VMEM((TAIL_MAIN,), jnp.int32),
        pltpu.VMEM((TAIL_LAST,), jnp.int32),
        pltpu.VMEM((NBUF, B, DC), jnp.float32),
        pltpu.VMEM((TAIL_MAIN, DC), jnp.float32),
        pltpu.VMEM((TAIL_LAST, DC), jnp.float32),
        pltpu.VMEM((GROWS, DC), jnp.float32),
        pltpu.VMEM_SHARED((NUM_GRAPHS, DC), jnp.float32),
    ] + [pltpu.SemaphoreType.DMA] * 5,
)
def _segsum_sc(h_hbm, gid_hbm, out_hbm,
               idx_all, idx_tm, idx_tl,
               rows_ring, rows_tm, rows_tl,
               zero_v, shared_acc, sem_i, *sems_ring):
    c = lax.axis_index("c")
    s = lax.axis_index("s")
    col0 = c * DC
    base = s * ROWS_PER_SUB
    sems = list(sems_ring)

    def h_blk(start):
        return h_hbm.at[pl.ds(start, B), pl.ds(col0, DC)]

    def fire(blk, l):
        pltpu.async_copy(h_blk(base + blk * B), rows_ring.at[l], sems[l])

    def drain(l):
        pltpu.make_async_copy(h_blk(base), rows_ring.at[l], sems[l]).wait()

    def fire_idx(j, _):
        pltpu.async_copy(gid_hbm.at[pl.ds(base + j * B, B)], idx_all.at[j],
                         sem_i)
        return 0

    lax.fori_loop(0, FULL_BLOCKS, fire_idx, 0)
    for l in range(NBUF):
        fire(l, l)

    zv = jnp.zeros((16,), jnp.float32)

    def zero_body(r, _):
        for j in range(DC // 16):
            zero_v[r, pl.ds(j * 16, 16)] = zv
        return 0

    lax.fori_loop(0, GROWS, zero_body, 0)
    pltpu.sync_copy(zero_v, shared_acc.at[pl.ds(s * GROWS, GROWS)])

    def drain_idx(j, _):
        pltpu.make_async_copy(gid_hbm.at[pl.ds(base + j * B, B)],
                              idx_all.at[j], sem_i).wait()
        return 0

    lax.fori_loop(0, FULL_BLOCKS, drain_idx, 0)

    plsc.subcore_barrier()

    def ring_body(p, _):
        blk0 = NBUF * p
        for l in range(NBUF):
            blk = blk0 + l
            drain(l)
            pltpu.sync_copy(rows_ring.at[l], shared_acc.at[idx_all.at[blk]],
                            add=True)

            @pl.when(blk + NBUF < FULL_BLOCKS)
            def _():
                fire(blk + NBUF, l)
        return 0

    lax.fori_loop(0, FULL_BLOCKS // NBUF, ring_body, 0)

    tail_start = base + FULL_BLOCKS * B

    @pl.when(s < NS - 1)
    def _():
        pltpu.sync_copy(gid_hbm.at[pl.ds(tail_start, TAIL_MAIN)], idx_tm)
        pltpu.sync_copy(h_hbm.at[pl.ds(tail_start, TAIL_MAIN), pl.ds(col0, DC)],
                        rows_tm)
        pltpu.sync_copy(rows_tm, shared_acc.at[idx_tm], add=True)

    @pl.when(s == NS - 1)
    def _():
        pltpu.sync_copy(gid_hbm.at[pl.ds(tail_start, TAIL_LAST)], idx_tl)
        pltpu.sync_copy(h_hbm.at[pl.ds(tail_start, TAIL_LAST), pl.ds(col0, DC)],
                        rows_tl)
        pltpu.sync_copy(rows_tl, shared_acc.at[idx_tl], add=True)

    plsc.subcore_barrier()

    pltpu.sync_copy(shared_acc.at[pl.ds(s * GROWS, GROWS)],
                    out_hbm.at[pl.ds(s * GROWS, GROWS), pl.ds(col0, DC)])


def kernel(h, graph_ids):
    return _segsum_sc(h, graph_ids.astype(jnp.int32))


# speedup vs baseline: 1.0180x; 1.0180x over previous
import functools

import jax
import jax.numpy as jnp
from jax import lax
from jax.experimental import pallas as pl
from jax.experimental.pallas import tpu as pltpu
from jax.experimental.pallas import tpu_sc as plsc

NUM_GRAPHS = 256
N_NODES = 100000
D_FEAT = 128

NC = 2
NS = 16
DC = D_FEAT // NC
B = 128
NBUF = 8
AHEAD = 4

ROWS_PER_SUB = 6256
LAST_ROWS = N_NODES - (NS - 1) * ROWS_PER_SUB
FULL_BLOCKS = 48
TAIL_MAIN = ROWS_PER_SUB - FULL_BLOCKS * B
TAIL_LAST = LAST_ROWS - FULL_BLOCKS * B
GROWS = NUM_GRAPHS // NS


@functools.partial(
    pl.kernel,
    mesh=plsc.VectorSubcoreMesh(core_axis_name="c", subcore_axis_name="s"),
    out_type=jax.ShapeDtypeStruct((NUM_GRAPHS, D_FEAT), jnp.float32),
    compiler_params=pltpu.CompilerParams(use_tc_tiling_on_sc=False),
    scratch_types=[
        pltpu.VMEM((FULL_BLOCKS, B), jnp.int32),
        pltpu.VMEM((TAIL_MAIN,), jnp.int32),
        pltpu.VMEM((TAIL_LAST,), jnp.int32),
        pltpu.VMEM((NBUF, B, DC), jnp.float32),
        pltpu.VMEM((TAIL_MAIN, DC), jnp.float32),
        pltpu.VMEM((TAIL_LAST, DC), jnp.float32),
        pltpu.VMEM((GROWS, DC), jnp.float32),
        pltpu.VMEM_SHARED((NUM_GRAPHS, DC), jnp.float32),
    ] + [pltpu.SemaphoreType.DMA] * 17,
)
def _segsum_sc(h_hbm, gid_hbm, out_hbm,
               idx_all, idx_tm, idx_tl,
               rows_ring, rows_tm, rows_tl,
               zero_v, shared_acc, sem_i, *sems_ring):
    c = lax.axis_index("c")
    s = lax.axis_index("s")
    col0 = c * DC
    base = s * ROWS_PER_SUB
    sems = list(sems_ring[:NBUF])
    sems_s = list(sems_ring[NBUF:])

    def h_blk(start):
        return h_hbm.at[pl.ds(start, B), pl.ds(col0, DC)]

    def fire(blk, l):
        pltpu.async_copy(h_blk(base + blk * B), rows_ring.at[l], sems[l])

    def drain(l):
        pltpu.make_async_copy(h_blk(base), rows_ring.at[l], sems[l]).wait()

    def fire_idx(j, _):
        pltpu.async_copy(gid_hbm.at[pl.ds(base + j * B, B)], idx_all.at[j],
                         sem_i)
        return 0

    lax.fori_loop(0, FULL_BLOCKS, fire_idx, 0)
    for l in range(AHEAD):
        fire(l, l)

    zv = jnp.zeros((16,), jnp.float32)

    def zero_body(r, _):
        for j in range(DC // 16):
            zero_v[r, pl.ds(j * 16, 16)] = zv
        return 0

    lax.fori_loop(0, GROWS, zero_body, 0)
    pltpu.sync_copy(zero_v, shared_acc.at[pl.ds(s * GROWS, GROWS)])

    def drain_idx(j, _):
        pltpu.make_async_copy(gid_hbm.at[pl.ds(base + j * B, B)],
                              idx_all.at[j], sem_i).wait()
        return 0

    lax.fori_loop(0, FULL_BLOCKS, drain_idx, 0)

    plsc.subcore_barrier()

    def scat(blk, l):
        pltpu.async_copy(rows_ring.at[l], shared_acc.at[idx_all.at[blk]],
                         sems_s[l], add=True)

    def scat_wait(l):
        pltpu.make_async_copy(rows_ring.at[l], shared_acc.at[idx_all.at[0]],
                              sems_s[l]).wait()

    def ring_body(p, _):
        blk0 = NBUF * p
        for i in range(NBUF):
            blk = blk0 + i
            drain(i)
            scat(blk, i)
            l2 = (i + AHEAD) % NBUF

            @pl.when(blk >= AHEAD)
            def _():
                scat_wait(l2)

            @pl.when(blk + AHEAD < FULL_BLOCKS)
            def _():
                fire(blk + AHEAD, l2)
        return 0

    lax.fori_loop(0, FULL_BLOCKS // NBUF, ring_body, 0)
    for l in range(NBUF - AHEAD, NBUF):
        scat_wait(l)

    tail_start = base + FULL_BLOCKS * B

    @pl.when(s < NS - 1)
    def _():
        pltpu.sync_copy(gid_hbm.at[pl.ds(tail_start, TAIL_MAIN)], idx_tm)
        pltpu.sync_copy(h_hbm.at[pl.ds(tail_start, TAIL_MAIN), pl.ds(col0, DC)],
                        rows_tm)
        pltpu.sync_copy(rows_tm, shared_acc.at[idx_tm], add=True)

    @pl.when(s == NS - 1)
    def _():
        pltpu.sync_copy(gid_hbm.at[pl.ds(tail_start, TAIL_LAST)], idx_tl)
        pltpu.sync_copy(h_hbm.at[pl.ds(tail_start, TAIL_LAST), pl.ds(col0, DC)],
                        rows_tl)
        pltpu.sync_copy(rows_tl, shared_acc.at[idx_tl], add=True)

    plsc.subcore_barrier()

    pltpu.sync_copy(shared_acc.at[pl.ds(s * GROWS, GROWS)],
                    out_hbm.at[pl.ds(s * GROWS, GROWS), pl.ds(col0, DC)])


def kernel(h, graph_ids):
    return _segsum_sc(h, graph_ids.astype(jnp.int32))


# speedup vs baseline: 1.5921x; 1.5640x over previous
import functools

import jax
import jax.numpy as jnp
from jax import lax
from jax.experimental import pallas as pl
from jax.experimental.pallas import tpu as pltpu
from jax.experimental.pallas import tpu_sc as plsc

NUM_GRAPHS = 256
N_NODES = 100000
D_FEAT = 128

NC = 2
NS = 16
DC = D_FEAT // NC
B = 128
NBUF = 4
L = 16

ROWS_PER_SUB = 6256
LAST_ROWS = N_NODES - (NS - 1) * ROWS_PER_SUB
FULL_BLOCKS = 48
TAIL_MAIN = ROWS_PER_SUB - FULL_BLOCKS * B
TAIL_LAST = LAST_ROWS - FULL_BLOCKS * B
GROWS = NUM_GRAPHS // NS


@functools.partial(
    pl.kernel,
    mesh=plsc.VectorSubcoreMesh(core_axis_name="c", subcore_axis_name="s"),
    out_type=jax.ShapeDtypeStruct((NUM_GRAPHS, D_FEAT), jnp.float32),
    compiler_params=pltpu.CompilerParams(use_tc_tiling_on_sc=False),
    scratch_types=[
        pltpu.VMEM((FULL_BLOCKS, B), jnp.int32),
        pltpu.VMEM((TAIL_MAIN,), jnp.int32),
        pltpu.VMEM((TAIL_LAST,), jnp.int32),
        pltpu.VMEM((NBUF, B, DC), jnp.float32),
        pltpu.VMEM((TAIL_MAIN, DC), jnp.float32),
        pltpu.VMEM((TAIL_LAST, DC), jnp.float32),
        pltpu.VMEM((NUM_GRAPHS, DC), jnp.float32),
        pltpu.VMEM((B,), jnp.int32),
        pltpu.VMEM((B,), jnp.int32),
        pltpu.VMEM_SHARED((NUM_GRAPHS, DC), jnp.float32),
    ] + [pltpu.SemaphoreType.DMA] * (NBUF + 1),
)
def _segsum_sc(h_hbm, gid_hbm, out_hbm,
               idx_all, idx_tm, idx_tl,
               rows_ring, rows_tm, rows_tl,
               acc_v, id_lo, id_hi, shared_acc, sem_i, *sems):
    c = lax.axis_index("c")
    s = lax.axis_index("s")
    col0 = c * DC
    base = s * ROWS_PER_SUB
    sems = list(sems)

    def h_blk(start):
        return h_hbm.at[pl.ds(start, B), pl.ds(col0, DC)]

    def fire(blk, l):
        pltpu.async_copy(h_blk(base + blk * B), rows_ring.at[l], sems[l])

    def drain(l):
        pltpu.make_async_copy(h_blk(base), rows_ring.at[l], sems[l]).wait()

    def fire_idx(j, _):
        pltpu.async_copy(gid_hbm.at[pl.ds(base + j * B, B)], idx_all.at[j],
                         sem_i)
        return 0

    lax.fori_loop(0, FULL_BLOCKS, fire_idx, 0)
    for l in range(NBUF):
        fire(l, l)

    zv = jnp.zeros((L,), jnp.float32)

    def zero_body(r, _):
        for j in range(DC // L):
            acc_v[r, pl.ds(j * L, L)] = zv
        return 0

    lax.fori_loop(0, NUM_GRAPHS, zero_body, 0)

    for j in range(B // L):
        iota = lax.iota(jnp.int32, L)
        id_lo[pl.ds(j * L, L)] = iota + (j * L)
        id_hi[pl.ds(j * L, L)] = iota + (j * L + B)

    @pl.when(s == 0)
    def _():
        pltpu.sync_copy(acc_v, shared_acc)

    def drain_idx(j, _):
        pltpu.make_async_copy(gid_hbm.at[pl.ds(base + j * B, B)],
                              idx_all.at[j], sem_i).wait()
        return 0

    lax.fori_loop(0, FULL_BLOCKS, drain_idx, 0)

    plsc.subcore_barrier()

    def ring_body(p, _):
        blk0 = NBUF * p
        for l in range(NBUF):
            blk = blk0 + l
            drain(l)

            vfirst = idx_all[blk, pl.ds(0, L)]
            vlast = idx_all[blk, pl.ds(B - L, L)]
            gmin = vfirst[0]
            gmax = vlast[L - 1]

            @pl.when(gmin == gmax)
            def _():
                def row_body(r, carry):
                    out = []
                    for j in range(DC // L):
                        out.append(carry[j] + rows_ring[l, r, pl.ds(j * L, L)])
                    return tuple(out)

                sums = lax.fori_loop(
                    0, B, row_body,
                    tuple(jnp.zeros((L,), jnp.float32)
                          for _ in range(DC // L)),
                    unroll=4)
                for j in range(DC // L):
                    acc_v[gmin, pl.ds(j * L, L)] = (
                        acc_v[gmin, pl.ds(j * L, L)] + sums[j])

            @pl.when(gmin != gmax)
            def _():
                pltpu.sync_copy(rows_ring.at[l],
                                shared_acc.at[idx_all.at[blk]], add=True)

            @pl.when(blk + NBUF < FULL_BLOCKS)
            def _():
                fire(blk + NBUF, l)
        return 0

    lax.fori_loop(0, FULL_BLOCKS // NBUF, ring_body, 0)

    tail_start = base + FULL_BLOCKS * B

    @pl.when(s < NS - 1)
    def _():
        pltpu.sync_copy(gid_hbm.at[pl.ds(tail_start, TAIL_MAIN)], idx_tm)
        pltpu.sync_copy(h_hbm.at[pl.ds(tail_start, TAIL_MAIN), pl.ds(col0, DC)],
                        rows_tm)
        pltpu.sync_copy(rows_tm, shared_acc.at[idx_tm], add=True)

    @pl.when(s == NS - 1)
    def _():
        pltpu.sync_copy(gid_hbm.at[pl.ds(tail_start, TAIL_LAST)], idx_tl)
        pltpu.sync_copy(h_hbm.at[pl.ds(tail_start, TAIL_LAST), pl.ds(col0, DC)],
                        rows_tl)
        pltpu.sync_copy(rows_tl, shared_acc.at[idx_tl], add=True)

    pltpu.sync_copy(acc_v.at[pl.ds(0, B)], shared_acc.at[id_lo], add=True)
    pltpu.sync_copy(acc_v.at[pl.ds(B, B)], shared_acc.at[id_hi], add=True)

    plsc.subcore_barrier()

    pltpu.sync_copy(shared_acc.at[pl.ds(s * GROWS, GROWS)],
                    out_hbm.at[pl.ds(s * GROWS, GROWS), pl.ds(col0, DC)])


def kernel(h, graph_ids):
    return _segsum_sc(h, graph_ids.astype(jnp.int32))
